# Initial kernel scaffold; baseline (speedup 1.0000x reference)
import jax
import jax.numpy as jnp
from jax.experimental import pallas as pl


def kernel(y_pred, y_true, ordinal_matrix):
    raise NotImplementedError("write your pallas kernel here")


# speedup vs baseline: 3.3154x; 3.3154x over previous
import functools

import jax
import jax.numpy as jnp
from jax import lax
from jax.experimental import pallas as pl
from jax.experimental.pallas import tpu as pltpu
from jax.experimental.pallas import tpu_sc as plsc

N_ROWS = 65536
C = 10
NC = 2
NS = 16
L = 16
NW = NC * NS
ROWS_PER_W = N_ROWS // NW
WORDS_PER_W = ROWS_PER_W * C
BLOCKS = ROWS_PER_W // L

_mesh = plsc.VectorSubcoreMesh(core_axis_name="c", subcore_axis_name="s")


@functools.partial(
    pl.kernel,
    out_type=jax.ShapeDtypeStruct((NW * L,), jnp.float32),
    mesh=_mesh,
    compiler_params=pltpu.CompilerParams(needs_layout_passes=False),
    scratch_types=[
        pltpu.VMEM((WORDS_PER_W,), jnp.float32),
        pltpu.VMEM((ROWS_PER_W,), jnp.int32),
        pltpu.VMEM((128,), jnp.float32),
        pltpu.VMEM((L,), jnp.float32),
    ],
)
def _sc_loss(yp_hbm, yt_hbm, m2_hbm, out_hbm, ypv, ytv, m2v, accv):
    wid = lax.axis_index("s") * NC + lax.axis_index("c")
    row0 = wid * ROWS_PER_W
    pltpu.sync_copy(yp_hbm.at[pl.ds(row0 * C, WORDS_PER_W)], ypv)
    pltpu.sync_copy(yt_hbm.at[pl.ds(row0, ROWS_PER_W)], ytv)
    pltpu.sync_copy(m2_hbm, m2v)

    stride = lax.iota(jnp.int32, L) * C

    def body(b, acc):
        yt = ytv[pl.ds(b * L, L)]
        wbase = yt * C
        rowbase = b * (L * C) + stride
        for j in range(C):
            v = plsc.load_gather(ypv, [rowbase + j])
            w = plsc.load_gather(m2v, [wbase + j])
            acc = acc + (v * v) * w
        return acc

    acc = lax.fori_loop(0, BLOCKS, body, jnp.zeros((L,), jnp.float32))
    accv[...] = acc
    pltpu.sync_copy(accv, out_hbm.at[pl.ds(wid * L, L)])


def kernel(y_pred, y_true, ordinal_matrix):
    m2 = (ordinal_matrix * ordinal_matrix).reshape(-1) / y_pred.shape[0]
    m2 = jnp.pad(m2, (0, 128 - m2.shape[0]))
    partials = _sc_loss(y_pred.reshape(-1), y_true, m2)
    return jnp.sum(partials)
